# baseline (speedup 1.0000x reference)
import functools

import jax
import jax.numpy as jnp
from jax import lax
from jax.experimental import pallas as pl
from jax.experimental.pallas import tpu as pltpu
from jax.experimental.pallas import tpu_sc as plsc

N_POS = 1024
SUB = 128
SIZE = 768
NSEG = 6
LANES = 16


def _sc_workers():
    try:
        info = plsc.get_sparse_core_info()
        return info.num_cores, info.num_subcores
    except Exception:
        return 2, 16


def kernel(xmin, ymin, xmax, ymax, width, height, first_page, last_page,
           x_table, y_table, w_table, h_table,
           first_page_embedding, last_page_embedding):
    B, L = xmin.shape
    NB = B * L
    NC, NS = _sc_workers()
    NW = NC * NS
    rows_per_w = NB // NW
    R = 128
    n_chunks = rows_per_w // R
    assert rows_per_w * NW == NB and n_chunks * R == rows_per_w

    coords = [a.reshape(NB) for a in (xmin, ymin, xmax, ymax, width, height)]
    fp = first_page.reshape(NB)
    lp = last_page.reshape(NB)

    mesh = plsc.VectorSubcoreMesh(core_axis_name="c", subcore_axis_name="s",
                                  num_cores=NC, num_subcores=NS)

    @functools.partial(
        pl.kernel,
        out_type=jax.ShapeDtypeStruct((NB, SIZE), jnp.float32),
        mesh=mesh,
        scratch_types=[
            pltpu.VMEM((NSEG, R), jnp.float32),
            pltpu.VMEM((R + LANES,), jnp.float32),
            pltpu.VMEM((R + LANES,), jnp.float32),
            pltpu.VMEM((NSEG, R), jnp.int32),
            pltpu.VMEM((NSEG, R, SUB), jnp.float32),
            pltpu.VMEM((2, SIZE), jnp.float32),
            pltpu.SemaphoreType.DMA,
            pltpu.SemaphoreType.DMA,
        ],
    )
    def sc_kernel(xt_h, yt_h, wt_h, ht_h,
                  xmin_h, ymin_h, xmax_h, ymax_h, w_h, h_h, fp_h, lp_h,
                  fpe_h, lpe_h, out_h,
                  in_v, fp_v, lp_v, idx_v, rows_v, pe_v, gsem, osem):
        wid = lax.axis_index("s") * NC + lax.axis_index("c")
        base_w = wid * rows_per_w
        pltpu.sync_copy(fpe_h, pe_v.at[0])
        pltpu.sync_copy(lpe_h, pe_v.at[1])
        coord_hs = [xmin_h, ymin_h, xmax_h, ymax_h, w_h, h_h]
        tables = [xt_h, yt_h, xt_h, yt_h, wt_h, ht_h]

        def chunk_body(c, carry):
            base = base_w + c * R
            for a in range(NSEG):
                pltpu.sync_copy(coord_hs[a].at[pl.ds(base, R)], in_v.at[a])
            pltpu.sync_copy(fp_h.at[pl.ds(base, R)], fp_v.at[pl.ds(0, R)])
            pltpu.sync_copy(lp_h.at[pl.ds(base, R)], lp_v.at[pl.ds(0, R)])
            for a in range(NSEG):
                for t in range(R // LANES):
                    sl = pl.ds(t * LANES, LANES)
                    v = in_v[a, sl]
                    if a == 5:
                        v = v * 5.0
                    v = jnp.minimum(v * float(N_POS), float(N_POS - 1))
                    idx_v[a, sl] = v.astype(jnp.int32)
            ghs = [pltpu.async_copy(tables[a].at[idx_v.at[a]], rows_v.at[a],
                                    gsem) for a in range(NSEG)]
            ohs = []
            for a in range(NSEG):
                ghs[a].wait()
                fpe8 = [pe_v[0, pl.ds(a * SUB + j * LANES, LANES)]
                        for j in range(SUB // LANES)]
                lpe8 = [pe_v[1, pl.ds(a * SUB + j * LANES, LANES)]
                        for j in range(SUB // LANES)]

                def row_body(i, _, a=a, fpe8=fpe8, lpe8=lpe8):
                    fpi = fp_v[pl.ds(i, LANES)][0]
                    lpi = lp_v[pl.ds(i, LANES)][0]
                    for j in range(SUB // LANES):
                        sl = pl.ds(j * LANES, LANES)
                        rows_v[a, i, sl] = (rows_v[a, i, sl]
                                            + fpi * fpe8[j] + lpi * lpe8[j])
                    return 0

                lax.fori_loop(0, R, row_body, 0)
                ohs.append(pltpu.async_copy(
                    rows_v.at[a],
                    out_h.at[pl.ds(base, R), pl.ds(a * SUB, SUB)], osem))
            for oh in ohs:
                oh.wait()
            return carry

        lax.fori_loop(0, n_chunks, chunk_body, 0)

    out = sc_kernel(x_table, y_table, w_table, h_table,
                    *coords, fp, lp,
                    first_page_embedding, last_page_embedding)
    return out.reshape(B, L, SIZE)


# speedup vs baseline: 1.0021x; 1.0021x over previous
import functools

import jax
import jax.numpy as jnp
from jax import lax
from jax.experimental import pallas as pl
from jax.experimental.pallas import tpu as pltpu
from jax.experimental.pallas import tpu_sc as plsc

N_POS = 1024
SUB = 128
SIZE = 768
NSEG = 6
LANES = 16


def _sc_workers():
    try:
        info = plsc.get_sparse_core_info()
        return info.num_cores, info.num_subcores
    except Exception:
        return 2, 16


def kernel(xmin, ymin, xmax, ymax, width, height, first_page, last_page,
           x_table, y_table, w_table, h_table,
           first_page_embedding, last_page_embedding):
    B, L = xmin.shape
    NB = B * L
    NC, NS = _sc_workers()
    NW = NC * NS
    rows_per_w = NB // NW
    R = 128
    n_chunks = rows_per_w // R
    assert rows_per_w * NW == NB and n_chunks * R == rows_per_w

    coords = [a.reshape(NB) for a in (xmin, ymin, xmax, ymax, width, height)]
    fp = first_page.reshape(NB)
    lp = last_page.reshape(NB)

    mesh = plsc.VectorSubcoreMesh(core_axis_name="c", subcore_axis_name="s",
                                  num_cores=NC, num_subcores=NS)

    @functools.partial(
        pl.kernel,
        out_type=jax.ShapeDtypeStruct((NB, SIZE), jnp.float32),
        mesh=mesh,
        scratch_types=[
            pltpu.VMEM((NSEG + 2, R), jnp.float32),
            pltpu.VMEM((NSEG, R), jnp.int32),
            pltpu.VMEM((NSEG, R, SUB), jnp.float32),
            pltpu.VMEM((2, SIZE), jnp.float32),
            pltpu.SemaphoreType.DMA,
            pltpu.SemaphoreType.DMA,
        ],
    )
    def sc_kernel(xt_h, yt_h, wt_h, ht_h,
                  xmin_h, ymin_h, xmax_h, ymax_h, w_h, h_h, fp_h, lp_h,
                  fpe_h, lpe_h, out_h,
                  in_v, idx_v, rows_v, pe_v, gsem, osem):
        lane_iota = lax.broadcasted_iota(jnp.int32, (LANES,), 0)
        lane_bcast = [lane_iota * 0 + u for u in range(LANES)]
        wid = lax.axis_index("s") * NC + lax.axis_index("c")
        base_w = wid * rows_per_w
        pltpu.sync_copy(fpe_h, pe_v.at[0])
        pltpu.sync_copy(lpe_h, pe_v.at[1])
        coord_hs = [xmin_h, ymin_h, xmax_h, ymax_h, w_h, h_h]
        tables = [xt_h, yt_h, xt_h, yt_h, wt_h, ht_h]

        def chunk_body(c, carry):
            base = base_w + c * R
            for a in range(NSEG):
                pltpu.sync_copy(coord_hs[a].at[pl.ds(base, R)], in_v.at[a])
            pltpu.sync_copy(fp_h.at[pl.ds(base, R)], in_v.at[NSEG])
            pltpu.sync_copy(lp_h.at[pl.ds(base, R)], in_v.at[NSEG + 1])
            for a in range(NSEG):
                for t in range(R // LANES):
                    sl = pl.ds(t * LANES, LANES)
                    v = in_v[a, sl]
                    if a == 5:
                        v = v * 5.0
                    v = jnp.minimum(v * float(N_POS), float(N_POS - 1))
                    idx_v[a, sl] = v.astype(jnp.int32)
            ghs = [pltpu.async_copy(tables[a].at[idx_v.at[a]], rows_v.at[a],
                                    gsem) for a in range(NSEG)]
            ohs = []
            for a in range(NSEG):
                ghs[a].wait()
                fpe8 = [pe_v[0, pl.ds(a * SUB + j * LANES, LANES)]
                        for j in range(SUB // LANES)]
                lpe8 = [pe_v[1, pl.ds(a * SUB + j * LANES, LANES)]
                        for j in range(SUB // LANES)]

                def row_body(g, _, a=a, fpe8=fpe8, lpe8=lpe8):
                    gsl = pl.ds(g * LANES, LANES)
                    fp16 = in_v[NSEG, gsl]
                    lp16 = in_v[NSEG + 1, gsl]
                    for u in range(LANES):
                        i = g * LANES + u
                        fpi = jnp.take_along_axis(fp16, lane_bcast[u], axis=0)
                        lpi = jnp.take_along_axis(lp16, lane_bcast[u], axis=0)
                        for j in range(SUB // LANES):
                            sl = pl.ds(j * LANES, LANES)
                            rows_v[a, i, sl] = (rows_v[a, i, sl]
                                                + fpi * fpe8[j]
                                                + lpi * lpe8[j])
                    return 0

                lax.fori_loop(0, R // LANES, row_body, 0)
                ohs.append(pltpu.async_copy(
                    rows_v.at[a],
                    out_h.at[pl.ds(base, R), pl.ds(a * SUB, SUB)], osem))
            for oh in ohs:
                oh.wait()
            return carry

        lax.fori_loop(0, n_chunks, chunk_body, 0)

    out = sc_kernel(x_table, y_table, w_table, h_table,
                    *coords, fp, lp,
                    first_page_embedding, last_page_embedding)
    return out.reshape(B, L, SIZE)
